# baseline (speedup 1.0000x reference)
import jax
import jax.numpy as jnp
from jax.experimental import pallas as pl

QS = 16384
NB_ROWS = 4096
ROW = 16 * 8 * 8
HALF = ROW // 2
BLK = 2048
GRID = QS // BLK
NBB = NB_ROWS // BLK


def _store_kernel(bfl_ref, bfr_ref, fl_ref, fr_ref, blab_ref, lab_ref,
                  out_ref, lab_out_ref):
    i = pl.program_id(0)

    @pl.when(i < NBB)
    def _():
        out_ref[:, 0:HALF] = bfl_ref[...]
        out_ref[:, HALF:] = bfr_ref[...]

    @pl.when(i >= NBB)
    def _():
        out_ref[:, 0:HALF] = fl_ref[...]
        out_ref[:, HALF:] = fr_ref[...]

    @pl.when(i == 0)
    def _():
        lab_out_ref[0:32, :] = blab_ref[...]
        lab_out_ref[32:, :] = lab_ref[32:, :]


def kernel(batch_features, batch_labels, features, labels):
    bf = batch_features.reshape(NB_ROWS, ROW)
    f = features.reshape(QS, ROW)
    bl = batch_labels.reshape(32, 128)
    lab = labels.reshape(128, 128)
    out, lab_out = pl.pallas_call(
        _store_kernel,
        grid=(GRID,),
        in_specs=[
            pl.BlockSpec((BLK, HALF), lambda i: (jnp.minimum(i, NBB - 1), 0)),
            pl.BlockSpec((BLK, HALF), lambda i: (jnp.minimum(i, NBB - 1), 1)),
            pl.BlockSpec((BLK, HALF), lambda i: (jnp.maximum(i, NBB), 0)),
            pl.BlockSpec((BLK, HALF), lambda i: (jnp.maximum(i, NBB), 1)),
            pl.BlockSpec((32, 128), lambda i: (0, 0)),
            pl.BlockSpec((128, 128), lambda i: (0, 0)),
        ],
        out_specs=[
            pl.BlockSpec((BLK, ROW), lambda i: (i, 0)),
            pl.BlockSpec((128, 128), lambda i: (0, 0)),
        ],
        out_shape=[
            jax.ShapeDtypeStruct((QS, ROW), jnp.float32),
            jax.ShapeDtypeStruct((128, 128), jnp.int32),
        ],
    )(bf, bf, f, f, bl, lab)
    return out.reshape(QS, 16, 8, 8), lab_out.reshape(QS)
